# baseline (speedup 1.0000x reference)
import functools

import jax
import jax.numpy as jnp
from jax import lax
from jax.experimental import pallas as pl
from jax.experimental.pallas import tpu as pltpu
from jax.experimental.pallas import tpu_sc as plsc

B_USER = 16384
B_ITEM = 4096
K = 32
L = 16
TW = 128

_INFO = plsc.get_sparse_core_info()
_NC = _INFO.num_cores
_NS = _INFO.num_subcores
_NW = _NC * _NS

_BU_PER = B_USER // _NW
_BI_PER = B_ITEM // _NW
_GRP = 8


def _extract_scalars(idx16, count):
    iota = lax.iota(jnp.int32, L)
    out = []
    for l in range(count):
        out.append(lax.reduce_max(
            jnp.where(iota == l, idx16, 0), axes=(0,)))
    return out


def _fire_group(tab, scalars, chunks_v, sems, buf):
    boff = pl.multiple_of(buf * (_GRP * K), K)
    sem = sems.at[buf]
    for t, idx in enumerate(scalars):
        c_al = pl.multiple_of((idx >> 7) * TW, TW)
        pltpu.async_copy(tab.at[:, pl.ds(c_al, TW)],
                         chunks_v.at[pl.ds(boff + t * K, K)], sem)


def _gather_one(tab, idx_v, out_rows_v, chunks_v, sems, n):
    ngroups = n // _GRP
    iota = lax.iota(jnp.int32, L)

    def load_scalars(g):
        vec = idx_v[pl.ds(pl.multiple_of((g // 2) * L, L), L)]
        half0 = _extract_scalars(vec, L)
        lo = lax.rem(g, 2) * _GRP
        return [lax.select(lo > 0, half0[t + _GRP], half0[t]) for t in range(_GRP)]

    def body(g, carry):
        buf = lax.rem(g, 2)
        boff = pl.multiple_of(buf * (_GRP * K), K)
        @pl.when(g + 1 < ngroups)
        def _():
            _fire_group(tab, load_scalars(g + 1), chunks_v, sems, 1 - buf)

        for t in range(_GRP):
            pltpu.make_async_copy(tab.at[:, pl.ds(0, TW)],
                                  chunks_v.at[pl.ds(boff + t * K, K)],
                                  sems.at[buf]).wait()

        scalars = load_scalars(g)
        brow = jnp.full((L,), boff, jnp.int32)
        for t, idx in enumerate(scalars):
            j = g * _GRP + t
            col = jnp.full((L,), idx & (TW - 1), jnp.int32)
            row_j = jnp.full((L,), j, jnp.int32)
            v0 = plsc.load_gather(chunks_v, [brow + (t * K) + iota, col])
            v1 = plsc.load_gather(chunks_v, [brow + (t * K + L) + iota, col])
            plsc.store_scatter(out_rows_v, [row_j, iota], v0)
            plsc.store_scatter(out_rows_v, [row_j, L + iota], v1)

        return carry

    _fire_group(tab, load_scalars(0), chunks_v, sems, 0)
    lax.fori_loop(0, ngroups, body, 0, unroll=False)


_CHUNK_SIZES = (4096, 10240, 2048)


def _make_uv_body(n_per):
    def body(user_hbm, item_hbm, ut_hbm, it_hbm,
             u_out, v_out, uidx_v, iidx_v, urows_v, irows_v,
             chunks_v, sems):
        wid = lax.axis_index("s") * _NC + lax.axis_index("c")
        pltpu.sync_copy(user_hbm.at[wid], uidx_v)
        pltpu.sync_copy(item_hbm.at[wid], iidx_v)
        _gather_one(ut_hbm, uidx_v, urows_v, chunks_v, sems, n_per)
        _gather_one(it_hbm, iidx_v, irows_v, chunks_v, sems, _BI_PER)
        pltpu.sync_copy(urows_v, u_out.at[pl.ds(wid * n_per, n_per)])
        pltpu.sync_copy(irows_v, v_out.at[pl.ds(wid * _BI_PER, _BI_PER)])
    return body


def _make_u_body(n_per):
    def body(user_hbm, ut_hbm, u_out, uidx_v, urows_v, chunks_v, sems):
        wid = lax.axis_index("s") * _NC + lax.axis_index("c")
        pltpu.sync_copy(user_hbm.at[wid], uidx_v)
        _gather_one(ut_hbm, uidx_v, urows_v, chunks_v, sems, n_per)
        pltpu.sync_copy(urows_v, u_out.at[pl.ds(wid * n_per, n_per)])
    return body


_MESH = plsc.VectorSubcoreMesh(core_axis_name="c", subcore_axis_name="s")
_SC_PARAMS = pltpu.CompilerParams(needs_layout_passes=False)


def _sc_gather_uv(user_w, item_w, ut_tab, it_tab, ch):
    n_per = ch // _NW
    f = functools.partial(
        pl.kernel,
        mesh=_MESH,
        out_type=[
            jax.ShapeDtypeStruct((ch, K), jnp.float32),
            jax.ShapeDtypeStruct((B_ITEM, K), jnp.float32),
        ],
        scratch_types=[
            pltpu.VMEM((n_per,), jnp.int32),
            pltpu.VMEM((_BI_PER,), jnp.int32),
            pltpu.VMEM((n_per, K), jnp.float32),
            pltpu.VMEM((_BI_PER, K), jnp.float32),
            pltpu.VMEM((2 * _GRP * K, TW), jnp.float32),
            pltpu.SemaphoreType.DMA((2,)),
        ],
        compiler_params=_SC_PARAMS,
    )(_make_uv_body(n_per))
    return f(user_w, item_w, ut_tab, it_tab)


def _sc_gather_u(user_w, ut_tab, ch):
    n_per = ch // _NW
    f = functools.partial(
        pl.kernel,
        mesh=_MESH,
        out_type=jax.ShapeDtypeStruct((ch, K), jnp.float32),
        scratch_types=[
            pltpu.VMEM((n_per,), jnp.int32),
            pltpu.VMEM((n_per, K), jnp.float32),
            pltpu.VMEM((2 * _GRP * K, TW), jnp.float32),
            pltpu.SemaphoreType.DMA((2,)),
        ],
        compiler_params=_SC_PARAMS,
    )(_make_u_body(n_per))
    return f(user_w, ut_tab)


_BM = 512


def _mm_body(u_ref, v_ref, o_ref):
    o_ref[...] = lax.dot_general(
        u_ref[...], v_ref[...],
        dimension_numbers=(((1,), (1,)), ((), ())),
        preferred_element_type=jnp.float32)


def _mm_body_alias(u_ref, v_ref, prev_ref, o_ref):
    del prev_ref
    o_ref[...] = lax.dot_general(
        u_ref[...], v_ref[...],
        dimension_numbers=(((1,), (1,)), ((), ())),
        preferred_element_type=jnp.float32)


def _tc_matmul_chunk(row0, ch, u_c, v, prev=None):
    blk0 = row0 // _BM
    cb = ch // _BM
    out_shape = jax.ShapeDtypeStruct((B_USER, B_ITEM), jnp.float32)
    out_spec = pl.BlockSpec((_BM, B_ITEM), lambda i, b=blk0: (b + i, 0))
    if prev is None:
        return pl.pallas_call(
            _mm_body,
            grid=(cb,),
            in_specs=[
                pl.BlockSpec((_BM, K), lambda i: (i, 0)),
                pl.BlockSpec((B_ITEM, K), lambda i: (0, 0)),
            ],
            out_specs=out_spec,
            out_shape=out_shape,
        )(u_c, v)
    return pl.pallas_call(
        _mm_body_alias,
        grid=(cb,),
        in_specs=[
            pl.BlockSpec((_BM, K), lambda i: (i, 0)),
            pl.BlockSpec((B_ITEM, K), lambda i: (0, 0)),
            pl.BlockSpec(memory_space=pl.ANY),
        ],
        out_specs=out_spec,
        out_shape=out_shape,
        input_output_aliases={2: 0},
    )(u_c, v, prev)


def kernel(user, item, emb_user, emb_item):
    user = user.astype(jnp.int32)
    item_w = item.astype(jnp.int32).reshape(_NW, _BI_PER)
    ut_tab = emb_user.T
    it_tab = emb_item.T
    offs = [0]
    for ch in _CHUNK_SIZES:
        offs.append(offs[-1] + ch)
    chunks = [user[offs[c]:offs[c + 1]].reshape(_NW, _CHUNK_SIZES[c] // _NW)
              for c in range(len(_CHUNK_SIZES))]
    u0, v = _sc_gather_uv(chunks[0], item_w, ut_tab, it_tab, _CHUNK_SIZES[0])
    us = [u0] + [_sc_gather_u(chunks[c], ut_tab, _CHUNK_SIZES[c])
                 for c in range(1, len(_CHUNK_SIZES))]
    out = _tc_matmul_chunk(0, _CHUNK_SIZES[0], us[0], v)
    for c in range(1, len(_CHUNK_SIZES)):
        out = _tc_matmul_chunk(offs[c], _CHUNK_SIZES[c], us[c], v, out)
    return out


# speedup vs baseline: 1.0039x; 1.0039x over previous
import functools

import jax
import jax.numpy as jnp
from jax import lax
from jax.experimental import pallas as pl
from jax.experimental.pallas import tpu as pltpu
from jax.experimental.pallas import tpu_sc as plsc

B_USER = 16384
B_ITEM = 4096
K = 32
L = 16
TW = 128

_INFO = plsc.get_sparse_core_info()
_NC = _INFO.num_cores
_NS = _INFO.num_subcores
_NW = _NC * _NS

_BU_PER = B_USER // _NW
_BI_PER = B_ITEM // _NW
_GRP = 8


def _extract_scalars(idx16, count):
    iota = lax.iota(jnp.int32, L)
    out = []
    for l in range(count):
        out.append(lax.reduce_max(
            jnp.where(iota == l, idx16, 0), axes=(0,)))
    return out


def _fire_group(tab, scalars, chunks_v, sems, buf):
    boff = pl.multiple_of(buf * (_GRP * K), K)
    sem = sems.at[buf]
    for t, idx in enumerate(scalars):
        c_al = pl.multiple_of((idx >> 7) * TW, TW)
        for kb in range(K // 8):
            pltpu.async_copy(
                tab.at[pl.ds(kb * 8, 8), pl.ds(c_al, TW)],
                chunks_v.at[pl.ds(boff + t * K + kb * 8, 8)], sem)


def _gather_one(tab, idx_v, out_rows_v, chunks_v, sems, n):
    ngroups = n // _GRP
    iota = lax.iota(jnp.int32, L)

    def load_scalars(g):
        vec = idx_v[pl.ds(pl.multiple_of((g // 2) * L, L), L)]
        half0 = _extract_scalars(vec, L)
        lo = lax.rem(g, 2) * _GRP
        return [lax.select(lo > 0, half0[t + _GRP], half0[t]) for t in range(_GRP)]

    def body(g, carry):
        buf = lax.rem(g, 2)
        boff = pl.multiple_of(buf * (_GRP * K), K)
        @pl.when(g + 1 < ngroups)
        def _():
            _fire_group(tab, load_scalars(g + 1), chunks_v, sems, 1 - buf)

        for t in range(_GRP):
            pltpu.make_async_copy(tab.at[:, pl.ds(0, TW)],
                                  chunks_v.at[pl.ds(boff + t * K, K)],
                                  sems.at[buf]).wait()

        scalars = load_scalars(g)
        brow = jnp.full((L,), boff, jnp.int32)
        for t, idx in enumerate(scalars):
            j = g * _GRP + t
            col = jnp.full((L,), idx & (TW - 1), jnp.int32)
            row_j = jnp.full((L,), j, jnp.int32)
            v0 = plsc.load_gather(chunks_v, [brow + (t * K) + iota, col])
            v1 = plsc.load_gather(chunks_v, [brow + (t * K + L) + iota, col])
            plsc.store_scatter(out_rows_v, [row_j, iota], v0)
            plsc.store_scatter(out_rows_v, [row_j, L + iota], v1)

        return carry

    _fire_group(tab, load_scalars(0), chunks_v, sems, 0)
    lax.fori_loop(0, ngroups, body, 0, unroll=False)


_CHUNK_SIZES = (4096, 9216, 3072)


def _make_uv_body(n_per):
    def body(user_hbm, item_hbm, ut_hbm, it_hbm,
             u_out, v_out, uidx_v, iidx_v, urows_v, irows_v,
             chunks_v, sems):
        wid = lax.axis_index("s") * _NC + lax.axis_index("c")
        pltpu.sync_copy(user_hbm.at[wid], uidx_v)
        pltpu.sync_copy(item_hbm.at[wid], iidx_v)
        _gather_one(ut_hbm, uidx_v, urows_v, chunks_v, sems, n_per)
        _gather_one(it_hbm, iidx_v, irows_v, chunks_v, sems, _BI_PER)
        pltpu.sync_copy(urows_v, u_out.at[pl.ds(wid * n_per, n_per)])
        pltpu.sync_copy(irows_v, v_out.at[pl.ds(wid * _BI_PER, _BI_PER)])
    return body


def _make_u_body(n_per):
    def body(user_hbm, ut_hbm, u_out, uidx_v, urows_v, chunks_v, sems):
        wid = lax.axis_index("s") * _NC + lax.axis_index("c")
        pltpu.sync_copy(user_hbm.at[wid], uidx_v)
        _gather_one(ut_hbm, uidx_v, urows_v, chunks_v, sems, n_per)
        pltpu.sync_copy(urows_v, u_out.at[pl.ds(wid * n_per, n_per)])
    return body


_MESH = plsc.VectorSubcoreMesh(core_axis_name="c", subcore_axis_name="s")
_SC_PARAMS = pltpu.CompilerParams(needs_layout_passes=False)


def _sc_gather_uv(user_w, item_w, ut_tab, it_tab, ch):
    n_per = ch // _NW
    f = functools.partial(
        pl.kernel,
        mesh=_MESH,
        out_type=[
            jax.ShapeDtypeStruct((ch, K), jnp.float32),
            jax.ShapeDtypeStruct((B_ITEM, K), jnp.float32),
        ],
        scratch_types=[
            pltpu.VMEM((n_per,), jnp.int32),
            pltpu.VMEM((_BI_PER,), jnp.int32),
            pltpu.VMEM((n_per, K), jnp.float32),
            pltpu.VMEM((_BI_PER, K), jnp.float32),
            pltpu.VMEM((2 * _GRP * K, TW), jnp.float32),
            pltpu.SemaphoreType.DMA((2,)),
        ],
        compiler_params=_SC_PARAMS,
    )(_make_uv_body(n_per))
    return f(user_w, item_w, ut_tab, it_tab)


def _sc_gather_u(user_w, ut_tab, ch):
    n_per = ch // _NW
    f = functools.partial(
        pl.kernel,
        mesh=_MESH,
        out_type=jax.ShapeDtypeStruct((ch, K), jnp.float32),
        scratch_types=[
            pltpu.VMEM((n_per,), jnp.int32),
            pltpu.VMEM((n_per, K), jnp.float32),
            pltpu.VMEM((2 * _GRP * K, TW), jnp.float32),
            pltpu.SemaphoreType.DMA((2,)),
        ],
        compiler_params=_SC_PARAMS,
    )(_make_u_body(n_per))
    return f(user_w, ut_tab)


_BM = 512


def _mm_body(u_ref, v_ref, o_ref):
    o_ref[...] = lax.dot_general(
        u_ref[...], v_ref[...],
        dimension_numbers=(((1,), (1,)), ((), ())),
        preferred_element_type=jnp.float32)


def _mm_body_alias(u_ref, v_ref, prev_ref, o_ref):
    del prev_ref
    o_ref[...] = lax.dot_general(
        u_ref[...], v_ref[...],
        dimension_numbers=(((1,), (1,)), ((), ())),
        preferred_element_type=jnp.float32)


def _tc_matmul_chunk(row0, ch, u_c, v, prev=None):
    blk0 = row0 // _BM
    cb = ch // _BM
    out_shape = jax.ShapeDtypeStruct((B_USER, B_ITEM), jnp.float32)
    out_spec = pl.BlockSpec((_BM, B_ITEM), lambda i, b=blk0: (b + i, 0))
    if prev is None:
        return pl.pallas_call(
            _mm_body,
            grid=(cb,),
            in_specs=[
                pl.BlockSpec((_BM, K), lambda i: (i, 0)),
                pl.BlockSpec((B_ITEM, K), lambda i: (0, 0)),
            ],
            out_specs=out_spec,
            out_shape=out_shape,
        )(u_c, v)
    return pl.pallas_call(
        _mm_body_alias,
        grid=(cb,),
        in_specs=[
            pl.BlockSpec((_BM, K), lambda i: (i, 0)),
            pl.BlockSpec((B_ITEM, K), lambda i: (0, 0)),
            pl.BlockSpec(memory_space=pl.ANY),
        ],
        out_specs=out_spec,
        out_shape=out_shape,
        input_output_aliases={2: 0},
    )(u_c, v, prev)


def kernel(user, item, emb_user, emb_item):
    user = user.astype(jnp.int32)
    item_w = item.astype(jnp.int32).reshape(_NW, _BI_PER)
    ut_tab = emb_user.T
    it_tab = emb_item.T
    offs = [0]
    for ch in _CHUNK_SIZES:
        offs.append(offs[-1] + ch)
    chunks = [user[offs[c]:offs[c + 1]].reshape(_NW, _CHUNK_SIZES[c] // _NW)
              for c in range(len(_CHUNK_SIZES))]
    u0, v = _sc_gather_uv(chunks[0], item_w, ut_tab, it_tab, _CHUNK_SIZES[0])
    us = [u0] + [_sc_gather_u(chunks[c], ut_tab, _CHUNK_SIZES[c])
                 for c in range(1, len(_CHUNK_SIZES))]
    out = _tc_matmul_chunk(0, _CHUNK_SIZES[0], us[0], v)
    for c in range(1, len(_CHUNK_SIZES)):
        out = _tc_matmul_chunk(offs[c], _CHUNK_SIZES[c], us[c], v, out)
    return out
